# baseline (speedup 1.0000x reference)
import functools

import jax
import jax.numpy as jnp
import numpy as np
from jax.experimental import pallas as pl
from jax.experimental.pallas import tpu as pltpu

B = 128
T = 8
P = 8
N = 32
D = 4
F = 128
H = 256
BB = 8


def _swarm_kernel(x_ref, c1w_ref, c1b_ref, c2w_ref, c2b_ref,
                  eew1_ref, eeb1_ref, eew2_ref, eeb2_ref,
                  new1_ref, neb1_ref, new2_ref, neb2_ref,
                  ndw1_ref, ndb1_ref, ndw2_ref, ndb2_ref,
                  outw_ref, outb_ref, o_ref):
    f32 = jnp.float32

    c1w = c1w_ref[...]
    c1b = c1b_ref[...]
    c2w = c2w_ref[...]
    c2b = c2b_ref[...]
    eeA = eew1_ref[:F]
    eeC = eew1_ref[F:]
    eeb1 = eeb1_ref[...]
    eew2 = eew2_ref[...]
    eeb2 = eeb2_ref[...]
    new1 = new1_ref[...]
    neb1 = neb1_ref[...]
    new2 = new2_ref[...]
    neb2 = neb2_ref[...]
    ndwp = ndw1_ref[:D]
    ndwm = ndw1_ref[D:]
    ndb1 = ndb1_ref[...]
    ndw2 = ndw2_ref[...]
    ndb2 = ndb2_ref[...]
    outw = outw_ref[...]
    outb = outb_ref[...]

    x = x_ref[...]
    HB = BB // 2

    dot = functools.partial(jnp.dot, preferred_element_type=f32)

    def make_state(xh):
        ts = [xh[:, t, :, :].reshape(HB * N, D) for t in range(T)]
        y1 = []
        for p in range(4):
            row = jnp.concatenate(ts[p:p + 5], axis=1)
            y1.append(jax.nn.relu(dot(row, c1w) + c1b))
        return {"ts": ts, "y1": y1, "preds": []}

    def step(st, i):
        ts, y1 = st["ts"], st["y1"]
        if i > 0:
            row = jnp.concatenate(ts[i + 3:i + 8], axis=1)
            y1.append(jax.nn.relu(dot(row, c1w) + c1b))
        y1cat = jnp.concatenate(y1[i:i + 4], axis=1)
        h = jax.nn.relu(dot(y1cat, c2w) + c2b)

        A = (dot(h, eeA) + eeb1).reshape(HB, N, 1, H)
        C = dot(h, eeC).reshape(HB, 1, N, H)
        x1 = jax.nn.relu(A + C)

        x2 = jax.nn.relu(dot(x1.reshape(HB * N * N, H), eew2) + eeb2)
        s_all = x2.reshape(HB, N, N, H).sum(axis=1)

        d1 = jax.nn.relu(A.reshape(HB * N, H) + C.reshape(HB * N, H))
        d2 = jax.nn.relu(dot(d1, eew2) + eeb2)
        node_msg = s_all.reshape(HB * N, H) - d2

        m = jax.nn.relu(dot(node_msg, new1) + neb1)
        m = jax.nn.relu(dot(m, new2) + neb2)

        prev = ts[i + T - 1]
        ns = jax.nn.relu(dot(prev, ndwp) + dot(m, ndwm) + ndb1)
        ns = jax.nn.relu(dot(ns, ndw2) + ndb2)
        nxt = dot(ns, outw) + outb + prev

        ts.append(nxt)
        st["preds"].append(nxt.reshape(HB, 1, N, D))

    st_a = make_state(x[:HB])
    st_b = make_state(x[HB:])
    for i in range(P):
        step(st_a, i)
        step(st_b, i)

    o_ref[...] = jnp.concatenate(
        [jnp.concatenate(st_a["preds"], axis=1),
         jnp.concatenate(st_b["preds"], axis=1)], axis=0)


def kernel(inputs, conv1_w, conv1_b, conv2_w, conv2_b, ee_w1, ee_b1, ee_w2,
           ee_b2, ne_w1, ne_b1, ne_w2, ne_b2, nd_w1, nd_b1, nd_w2, nd_b2,
           out_w, out_b):
    c1w = conv1_w.reshape(5 * D, F)
    c2w = conv2_w.reshape(4 * F, F)
    r2 = lambda b: b.reshape(1, -1)

    grid = (B // BB,)
    full = lambda a: pl.BlockSpec(a.shape, lambda i: (0,) * a.ndim)

    weights = [c1w, r2(conv1_b), c2w, r2(conv2_b),
               ee_w1, r2(ee_b1), ee_w2, r2(ee_b2),
               ne_w1, r2(ne_b1), ne_w2, r2(ne_b2),
               nd_w1, r2(nd_b1), nd_w2, r2(nd_b2),
               out_w, r2(out_b)]

    out = pl.pallas_call(
        _swarm_kernel,
        grid=grid,
        in_specs=[pl.BlockSpec((BB, T, N, D), lambda i: (i, 0, 0, 0))] +
                 [full(w) for w in weights],
        out_specs=pl.BlockSpec((BB, P, N, D), lambda i: (i, 0, 0, 0)),
        out_shape=jax.ShapeDtypeStruct((B, P, N, D), jnp.float32),
        compiler_params=pltpu.CompilerParams(
            dimension_semantics=("parallel",)),
    )(inputs, *weights)
    return out


# speedup vs baseline: 1.2330x; 1.2330x over previous
import functools

import jax
import jax.numpy as jnp
import numpy as np
from jax.experimental import pallas as pl
from jax.experimental.pallas import tpu as pltpu

B = 128
T = 8
P = 8
N = 32
D = 4
F = 128
H = 256
BB = 8


def _swarm_kernel(x_ref, c1w_ref, c1b_ref, c2w_ref, c2b_ref,
                  eew1_ref, eeb1_ref, eew2_ref, eeb2_ref,
                  new1_ref, neb1_ref, new2_ref, neb2_ref,
                  ndw1_ref, ndb1_ref, ndw2_ref, ndb2_ref,
                  outw_ref, outb_ref, o_ref):
    f32 = jnp.float32

    c1w = c1w_ref[...]
    c1b = c1b_ref[...]
    c2w = c2w_ref[...]
    c2b = c2b_ref[...]
    eeA = eew1_ref[:F]
    eeC = eew1_ref[F:]
    eeb1 = eeb1_ref[...]
    eew2 = eew2_ref[...]
    eeb2 = eeb2_ref[...]
    new1 = new1_ref[...]
    neb1 = neb1_ref[...]
    new2 = new2_ref[...]
    neb2 = neb2_ref[...]
    ndwp = ndw1_ref[:D]
    ndwm = ndw1_ref[D:]
    ndb1 = ndb1_ref[...]
    ndw2 = ndw2_ref[...]
    ndb2 = ndb2_ref[...]
    outw = outw_ref[...]
    outb = outb_ref[...]

    x = x_ref[...]

    ts = [x[:, t, :, :].reshape(BB * N, D) for t in range(T)]

    dot = functools.partial(jnp.dot, preferred_element_type=f32)

    y1 = []
    for p in range(4):
        row = jnp.concatenate(ts[p:p + 5], axis=1)
        y1.append(jax.nn.relu(dot(row, c1w) + c1b))

    preds = []
    for i in range(P):
        if i > 0:
            row = jnp.concatenate(ts[i + 3:i + 8], axis=1)
            y1.append(jax.nn.relu(dot(row, c1w) + c1b))
        y1cat = jnp.concatenate(y1[i:i + 4], axis=1)
        h = jax.nn.relu(dot(y1cat, c2w) + c2b)

        A = (dot(h, eeA) + eeb1).reshape(BB, N, 1, H)
        C = dot(h, eeC).reshape(BB, 1, N, H)
        x1 = jax.nn.relu(A + C)

        x2 = jax.nn.relu(dot(x1.reshape(BB * N * N, H), eew2) + eeb2)
        s_all = x2.reshape(BB, N, N, H).sum(axis=1)

        d1 = jax.nn.relu(A.reshape(BB * N, H) + C.reshape(BB * N, H))
        d2 = jax.nn.relu(dot(d1, eew2) + eeb2)
        node_msg = s_all.reshape(BB * N, H) - d2

        m = jax.nn.relu(dot(node_msg, new1) + neb1)
        m = jax.nn.relu(dot(m, new2) + neb2)

        prev = ts[i + T - 1]
        ns = jax.nn.relu(dot(prev, ndwp) + dot(m, ndwm) + ndb1)
        ns = jax.nn.relu(dot(ns, ndw2) + ndb2)
        nxt = dot(ns, outw) + outb + prev

        ts.append(nxt)
        preds.append(nxt.reshape(BB, 1, N, D))

    o_ref[...] = jnp.concatenate(preds, axis=1)


def kernel(inputs, conv1_w, conv1_b, conv2_w, conv2_b, ee_w1, ee_b1, ee_w2,
           ee_b2, ne_w1, ne_b1, ne_w2, ne_b2, nd_w1, nd_b1, nd_w2, nd_b2,
           out_w, out_b):
    c1w = conv1_w.reshape(5 * D, F)
    c2w = conv2_w.reshape(4 * F, F)
    r2 = lambda b: b.reshape(1, -1)

    grid = (B // BB,)
    full = lambda a: pl.BlockSpec(a.shape, lambda i: (0,) * a.ndim)

    weights = [c1w, r2(conv1_b), c2w, r2(conv2_b),
               ee_w1, r2(ee_b1), ee_w2, r2(ee_b2),
               ne_w1, r2(ne_b1), ne_w2, r2(ne_b2),
               nd_w1, r2(nd_b1), nd_w2, r2(nd_b2),
               out_w, r2(out_b)]

    out = pl.pallas_call(
        _swarm_kernel,
        grid=grid,
        in_specs=[pl.BlockSpec((BB, T, N, D), lambda i: (i, 0, 0, 0))] +
                 [full(w) for w in weights],
        out_specs=pl.BlockSpec((BB, P, N, D), lambda i: (i, 0, 0, 0)),
        out_shape=jax.ShapeDtypeStruct((B, P, N, D), jnp.float32),
        compiler_params=pltpu.CompilerParams(
            dimension_semantics=("arbitrary",)),
    )(inputs, *weights)
    return out


# speedup vs baseline: 1.4713x; 1.1933x over previous
import functools

import jax
import jax.numpy as jnp
import numpy as np
from jax.experimental import pallas as pl
from jax.experimental.pallas import tpu as pltpu

B = 128
T = 8
P = 8
N = 32
D = 4
F = 128
H = 256
BB = 8


def _swarm_kernel(x_ref, c1w_ref, c1b_ref, c2w_ref, c2b_ref,
                  eew1_ref, eeb1_ref, eew2_ref, eeb2_ref,
                  new1_ref, neb1_ref, new2_ref, neb2_ref,
                  ndw1_ref, ndb1_ref, ndw2_ref, ndb2_ref,
                  outw_ref, outb_ref, o_ref):
    f32 = jnp.float32

    c1w = c1w_ref[...]
    c1b = c1b_ref[...]
    c2w = c2w_ref[...]
    c2b = c2b_ref[...]
    eeA = eew1_ref[:F]
    eeC = eew1_ref[F:]
    eeb1 = eeb1_ref[...]
    eew2 = eew2_ref[...]
    eeb2 = eeb2_ref[...]
    new1 = new1_ref[...]
    neb1 = neb1_ref[...]
    new2 = new2_ref[...]
    neb2 = neb2_ref[...]
    ndwp = ndw1_ref[:D]
    ndwm = ndw1_ref[D:]
    ndb1 = ndb1_ref[...]
    ndw2 = ndw2_ref[...]
    ndb2 = ndb2_ref[...]
    outw = outw_ref[...]
    outb = outb_ref[...]

    x = x_ref[...]

    ts = [x[:, t, :, :].reshape(BB * N, D) for t in range(T)]

    dot = functools.partial(jnp.dot, preferred_element_type=f32)

    y1 = []
    for p in range(4):
        row = jnp.concatenate(ts[p:p + 5], axis=1)
        y1.append(jax.nn.relu(dot(row, c1w) + c1b))

    preds = []
    for i in range(P):
        if i > 0:
            row = jnp.concatenate(ts[i + 3:i + 8], axis=1)
            y1.append(jax.nn.relu(dot(row, c1w) + c1b))
        y1cat = jnp.concatenate(y1[i:i + 4], axis=1)
        h = jax.nn.relu(dot(y1cat, c2w) + c2b)

        A = (dot(h, eeA) + eeb1).reshape(BB, N, 1, H)
        C = dot(h, eeC).reshape(BB, 1, N, H)

        NC = N // 2
        s_all = None
        for c in range(2):
            a_c = A[:, c * NC:(c + 1) * NC]
            x1_c = jax.nn.relu(a_c + C)
            x2_c = jax.nn.relu(dot(x1_c.reshape(BB * NC * N, H), eew2)
                               + eeb2)
            part = x2_c.reshape(BB, NC, N, H).sum(axis=1)
            s_all = part if s_all is None else s_all + part

        d1 = jax.nn.relu(A.reshape(BB * N, H) + C.reshape(BB * N, H))
        d2 = jax.nn.relu(dot(d1, eew2) + eeb2)
        node_msg = s_all.reshape(BB * N, H) - d2

        m = jax.nn.relu(dot(node_msg, new1) + neb1)
        m = jax.nn.relu(dot(m, new2) + neb2)

        prev = ts[i + T - 1]
        ns = jax.nn.relu(dot(prev, ndwp) + dot(m, ndwm) + ndb1)
        ns = jax.nn.relu(dot(ns, ndw2) + ndb2)
        nxt = dot(ns, outw) + outb + prev

        ts.append(nxt)
        preds.append(nxt.reshape(BB, 1, N, D))

    o_ref[...] = jnp.concatenate(preds, axis=1)


def kernel(inputs, conv1_w, conv1_b, conv2_w, conv2_b, ee_w1, ee_b1, ee_w2,
           ee_b2, ne_w1, ne_b1, ne_w2, ne_b2, nd_w1, nd_b1, nd_w2, nd_b2,
           out_w, out_b):
    c1w = conv1_w.reshape(5 * D, F)
    c2w = conv2_w.reshape(4 * F, F)
    r2 = lambda b: b.reshape(1, -1)

    grid = (B // BB,)
    full = lambda a: pl.BlockSpec(a.shape, lambda i: (0,) * a.ndim)

    weights = [c1w, r2(conv1_b), c2w, r2(conv2_b),
               ee_w1, r2(ee_b1), ee_w2, r2(ee_b2),
               ne_w1, r2(ne_b1), ne_w2, r2(ne_b2),
               nd_w1, r2(nd_b1), nd_w2, r2(nd_b2),
               out_w, r2(out_b)]

    out = pl.pallas_call(
        _swarm_kernel,
        grid=grid,
        in_specs=[pl.BlockSpec((BB, T, N, D), lambda i: (i, 0, 0, 0))] +
                 [full(w) for w in weights],
        out_specs=pl.BlockSpec((BB, P, N, D), lambda i: (i, 0, 0, 0)),
        out_shape=jax.ShapeDtypeStruct((B, P, N, D), jnp.float32),
        compiler_params=pltpu.CompilerParams(
            dimension_semantics=("arbitrary",)),
    )(inputs, *weights)
    return out
